# baseline (speedup 1.0000x reference)
import jax
import jax.numpy as jnp
from jax.experimental import pallas as pl
from jax.experimental.pallas import tpu as pltpu

N_TOK = 1024
D_FEAT = 256
K_TOP = 64


def _attn_topk_kernel(x_ref, wk_ref, wq_ref, out_ref, p_ref):
    xb = x_ref[0]
    wk = wk_ref[...]
    wq = wq_ref[...]

    kproj = jnp.dot(xb, wk, preferred_element_type=jnp.float32)
    qproj = jnp.dot(xb, wq, preferred_element_type=jnp.float32)

    h = jax.lax.dot_general(
        kproj, qproj,
        dimension_numbers=(((1,), (1,)), ((), ())),
        preferred_element_type=jnp.float32,
    )

    scale = 1.0 / jnp.sqrt(jnp.float32(D_FEAT))
    g = scale * h
    m = jnp.max(g, axis=1, keepdims=True)
    e = jnp.exp(g - m)
    s = jnp.sum(e, axis=1, keepdims=True)
    a = jnp.sum(e / s, axis=0, keepdims=True)

    lane = jax.lax.broadcasted_iota(jnp.int32, (1, N_TOK), 1)

    def body(r, a_cur):
        mx = jnp.max(a_cur, axis=1, keepdims=True)
        is_mx = a_cur == mx
        idx = jnp.min(
            jnp.where(is_mx, lane, jnp.int32(N_TOK)), axis=1, keepdims=True
        )
        sel = lane == idx
        p_ref[pl.ds(r, 1), :] = sel.astype(jnp.float32)
        return jnp.where(sel, -jnp.inf, a_cur)

    jax.lax.fori_loop(0, K_TOP, body, a)

    out_ref[0] = jnp.dot(p_ref[...], xb, preferred_element_type=jnp.float32)


def kernel(x, wk, wq):
    N, T, n, d_in = x.shape
    xf = x.reshape(N * T, n, d_in)
    out = pl.pallas_call(
        _attn_topk_kernel,
        grid=(N * T,),
        in_specs=[
            pl.BlockSpec((1, n, d_in), lambda i: (i, 0, 0)),
            pl.BlockSpec((d_in, wk.shape[1]), lambda i: (0, 0)),
            pl.BlockSpec((d_in, wq.shape[1]), lambda i: (0, 0)),
        ],
        out_specs=pl.BlockSpec((1, K_TOP, d_in), lambda i: (i, 0, 0)),
        out_shape=jax.ShapeDtypeStruct((N * T, K_TOP, d_in), jnp.float32),
        scratch_shapes=[pltpu.VMEM((K_TOP, n), jnp.float32)],
        compiler_params=pltpu.CompilerParams(
            dimension_semantics=("arbitrary",),
        ),
    )(xf, wk, wq)
    return out.reshape(N, T, K_TOP, d_in)


# speedup vs baseline: 4.4971x; 4.4971x over previous
import jax
import jax.numpy as jnp
from jax.experimental import pallas as pl
from jax.experimental.pallas import tpu as pltpu

N_TOK = 1024
D_FEAT = 256
K_TOP = 64


def _attn_topk_kernel(x_ref, wk_ref, wq_ref, out_ref):
    xb = x_ref[0]
    wk = wk_ref[...]
    wq = wq_ref[...]

    kproj = jnp.dot(xb, wk, preferred_element_type=jnp.float32)
    qproj = jnp.dot(xb, wq, preferred_element_type=jnp.float32)

    h = jax.lax.dot_general(
        kproj, qproj,
        dimension_numbers=(((1,), (1,)), ((), ())),
        preferred_element_type=jnp.float32,
    )

    scale = 1.0 / jnp.sqrt(jnp.float32(D_FEAT))
    g = scale * h
    m = jnp.max(g, axis=1, keepdims=True)
    e = jnp.exp(g - m)
    ones_col = jnp.ones((N_TOK, 1), jnp.float32)
    s = jnp.dot(e, ones_col, preferred_element_type=jnp.float32)
    contrib = e * (1.0 / s)
    ones_row = jnp.ones((1, N_TOK), jnp.float32)
    a = jnp.dot(ones_row, contrib, preferred_element_type=jnp.float32)

    a_col = jnp.transpose(a)
    i_col = jax.lax.broadcasted_iota(jnp.int32, (N_TOK, 1), 0)
    i_row = jax.lax.broadcasted_iota(jnp.int32, (1, N_TOK), 1)
    cmp = ((a_col > a) | ((a_col == a) & (i_col < i_row))).astype(jnp.float32)
    rank = jnp.dot(ones_row, cmp, preferred_element_type=jnp.float32)
    rank_i = rank.astype(jnp.int32)
    r_iota = jax.lax.broadcasted_iota(jnp.int32, (K_TOP, N_TOK), 0)
    p = (r_iota == rank_i).astype(jnp.float32)

    out_ref[0] = jnp.dot(p, xb, preferred_element_type=jnp.float32)


def kernel(x, wk, wq):
    N, T, n, d_in = x.shape
    xf = x.reshape(N * T, n, d_in)
    out = pl.pallas_call(
        _attn_topk_kernel,
        grid=(N * T,),
        in_specs=[
            pl.BlockSpec((1, n, d_in), lambda i: (i, 0, 0)),
            pl.BlockSpec((d_in, wk.shape[1]), lambda i: (0, 0)),
            pl.BlockSpec((d_in, wq.shape[1]), lambda i: (0, 0)),
        ],
        out_specs=pl.BlockSpec((1, K_TOP, d_in), lambda i: (i, 0, 0)),
        out_shape=jax.ShapeDtypeStruct((N * T, K_TOP, d_in), jnp.float32),
        compiler_params=pltpu.CompilerParams(
            dimension_semantics=("arbitrary",),
        ),
    )(xf, wk, wq)
    return out.reshape(N, T, K_TOP, d_in)
